# baseline (speedup 1.0000x reference)
import functools

import jax
import jax.numpy as jnp
from jax import lax
from jax.experimental import pallas as pl
from jax.experimental.pallas import tpu as pltpu
from jax.experimental.pallas import tpu_sc as plsc

N = 10000
E = 320000
D = 128
NC = 2
NS = 16
NW = NC * NS
CHUNK = 80
NCHUNK = 128
EPW = NCHUNK * CHUNK
EP = NW * EPW
NPAD = 10240
RPT = NPAD // NS


def _sc_agg_body(with_deg, h_hbm, src_hbm, dst_hbm, *refs):
    if with_deg:
        (ain_hbm, aout_hbm, din_hbm, dout_hbm, acc_sh,
         si0, si1, si2, si3, di0, di1, di2, di3, rows0, rows1,
         semi0, semi1, semi2, semi3, semg0, semg1, sems0, sems1) = refs
    else:
        (ain_hbm, aout_hbm, acc_sh,
         si0, si1, si2, si3, di0, di1, di2, di3, rows0, rows1,
         semi0, semi1, semi2, semi3, semg0, semg1, sems0, sems1) = refs

    sidxs = (si0, si1, si2, si3)
    didxs = (di0, di1, di2, di3)
    rows = (rows0, rows1)
    semi = (semi0, semi1, semi2, semi3)
    semg = (semg0, semg1)
    sems = (sems0, sems1)

    cid = lax.axis_index("c")
    sid = lax.axis_index("s")
    wid = cid * NS + sid
    r0 = sid * RPT

    def _fill_rows(val):
        vv = jnp.full((16,), val, jnp.float32)

        def _fill(i, c):
            for jj in range(D // 16):
                rows0[i, pl.ds(jj * 16, 16)] = vv
            return c
        lax.fori_loop(0, CHUNK, _fill, 0)

    def _zero_acc():
        _fill_rows(0.0)
        for k in range(RPT // CHUNK):
            pltpu.sync_copy(rows0, acc_sh.at[pl.ds(r0 + k * CHUNK, CHUNK)])

    def _accumulate(gsel_hbm, ssel_hbm):
        def start_idx(k, ib):
            pltpu.async_copy(gsel_hbm.at[wid, k], sidxs[ib], semi[ib])
            pltpu.async_copy(ssel_hbm.at[wid, k], didxs[ib], semi[ib])

        def wait_idx(ib):
            pltpu.make_async_copy(gsel_hbm.at[wid, 0], sidxs[ib],
                                  semi[ib]).wait()
            pltpu.make_async_copy(ssel_hbm.at[wid, 0], didxs[ib],
                                  semi[ib]).wait()

        def start_gather(b, ib):
            pltpu.async_copy(h_hbm.at[sidxs[ib]], rows[b], semg[b])

        def wait_gather(b):
            pltpu.make_async_copy(h_hbm.at[sidxs[0]], rows[b],
                                  semg[b]).wait()

        def start_scatter(b, ib):
            pltpu.async_copy(rows[b], acc_sh.at[didxs[ib]], sems[b],
                             add=True)

        def wait_scatter(b, ib):
            pltpu.make_async_copy(rows[b], acc_sh.at[didxs[ib]],
                                  sems[b]).wait()

        def sub(k, t, with_idx=True):
            wait_idx((t + 1) % 4)
            wait_scatter((t + 1) % 2, (t + 3) % 4)
            if with_idx:
                start_idx(k + 2, (t + 2) % 4)
            start_gather((t + 1) % 2, (t + 1) % 4)
            wait_gather(t % 2)
            start_scatter(t % 2, t % 4)

        start_idx(0, 0)
        start_idx(1, 1)
        wait_idx(0)
        start_gather(0, 0)
        wait_idx(1)
        start_idx(2, 2)
        start_gather(1, 1)
        wait_gather(0)
        start_scatter(0, 0)

        def _body(i, c):
            k = 4 * i + 1
            sub(k, 1)
            sub(k + 1, 2)
            sub(k + 2, 3)
            sub(k + 3, 0)
            return c
        lax.fori_loop(0, (NCHUNK - 4) // 4, _body, 0)
        sub(NCHUNK - 3, 1)
        sub(NCHUNK - 2, 2, with_idx=False)
        wait_scatter(0, 2)
        wait_gather(1)
        start_scatter(1, 3)
        wait_scatter(1, 3)

    def _count(ssel_hbm):
        def start_idx(k, ib):
            pltpu.async_copy(ssel_hbm.at[wid, k], didxs[ib], semi[ib])

        def wait_idx(ib):
            pltpu.make_async_copy(ssel_hbm.at[wid, 0], didxs[ib],
                                  semi[ib]).wait()

        def start_scatter(b, ib):
            pltpu.async_copy(rows0, acc_sh.at[didxs[ib]], sems[b], add=True)

        def wait_scatter(b, ib):
            pltpu.make_async_copy(rows0, acc_sh.at[didxs[ib]],
                                  sems[b]).wait()

        def sub(k, t, with_idx=True):
            wait_scatter((t + 1) % 2, (t + 3) % 4)
            if with_idx:
                start_idx(k + 2, (t + 2) % 4)
            wait_idx(t % 4)
            start_scatter(t % 2, t % 4)

        start_idx(0, 0)
        start_idx(1, 1)
        start_idx(2, 2)
        wait_idx(0)
        start_scatter(0, 0)
        sub(1, 1, with_idx=True)

        def _body(i, c):
            k = 4 * i + 2
            sub(k, 2)
            sub(k + 1, 3)
            sub(k + 2, 0)
            sub(k + 3, 1)
            return c
        lax.fori_loop(0, (NCHUNK - 4) // 4, _body, 0)
        sub(NCHUNK - 2, 2, with_idx=False)
        sub(NCHUNK - 1, 3, with_idx=False)
        wait_scatter(1, 3)

    def _writeout(out_hbm):
        pltpu.sync_copy(acc_sh.at[pl.ds(r0, RPT)],
                        out_hbm.at[cid, pl.ds(r0, RPT)])

    _zero_acc()
    plsc.subcore_barrier()
    _accumulate(src_hbm, dst_hbm)
    plsc.subcore_barrier()
    _writeout(ain_hbm)
    _zero_acc()
    plsc.subcore_barrier()

    _accumulate(dst_hbm, src_hbm)
    plsc.subcore_barrier()
    _writeout(aout_hbm)

    if with_deg:
        _zero_acc()
        _fill_rows(1.0)
        plsc.subcore_barrier()
        _count(dst_hbm)
        plsc.subcore_barrier()
        _writeout(din_hbm)
        _zero_acc()
        _fill_rows(1.0)
        plsc.subcore_barrier()
        _count(src_hbm)
        plsc.subcore_barrier()
        _writeout(dout_hbm)


def _make_sc_agg(with_deg):
    n_out = 4 if with_deg else 2
    out_type = tuple(jax.ShapeDtypeStruct((NC, NPAD, D), jnp.float32)
                     for _ in range(n_out))
    scratch = ([pltpu.VMEM_SHARED((NPAD, D), jnp.float32)]
               + [pltpu.VMEM((CHUNK,), jnp.int32) for _ in range(8)]
               + [pltpu.VMEM((CHUNK, D), jnp.float32) for _ in range(2)]
               + [pltpu.SemaphoreType.DMA for _ in range(8)])
    mesh = plsc.VectorSubcoreMesh(core_axis_name="c", subcore_axis_name="s")
    return pl.kernel(functools.partial(_sc_agg_body, with_deg),
                     out_type=out_type, mesh=mesh,
                     scratch_types=scratch)


_sc_agg_deg = _make_sc_agg(True)
_sc_agg = _make_sc_agg(False)

BN = 512
GRID = NPAD // BN


def _fuse_body(h_ref, ain_ref, aout_ref, din_ref, dout_ref,
               ws_ref, wni_ref, wno_ref, wg_ref, out_ref):
    f32 = jnp.float32
    h = h_ref[...]
    din = jnp.maximum(din_ref[0, :, 0:1] + din_ref[1, :, 0:1], 1.0)
    dout = jnp.maximum(dout_ref[0, :, 0:1] + dout_ref[1, :, 0:1], 1.0)
    a_in = (ain_ref[0] + ain_ref[1]) / din
    a_out = (aout_ref[0] + aout_ref[1]) / dout
    hs = jnp.dot(h, ws_ref[...], preferred_element_type=f32)
    h_in = hs + jnp.dot(a_in, wni_ref[...], preferred_element_type=f32)
    h_out = hs + jnp.dot(a_out, wno_ref[...], preferred_element_type=f32)
    g = (jnp.dot(h_in, wg_ref[0:D, :], preferred_element_type=f32)
         + jnp.dot(h_out, wg_ref[D:2 * D, :], preferred_element_type=f32))
    z = jax.nn.sigmoid(g)
    out_ref[...] = jnp.maximum(z * h_in + (1.0 - z) * h_out, 0.0)


_fuse = pl.pallas_call(
    _fuse_body,
    grid=(GRID,),
    in_specs=[
        pl.BlockSpec((BN, D), lambda i: (i, 0)),
        pl.BlockSpec((NC, BN, D), lambda i: (0, i, 0)),
        pl.BlockSpec((NC, BN, D), lambda i: (0, i, 0)),
        pl.BlockSpec((NC, BN, D), lambda i: (0, i, 0)),
        pl.BlockSpec((NC, BN, D), lambda i: (0, i, 0)),
        pl.BlockSpec((D, D), lambda i: (0, 0)),
        pl.BlockSpec((D, D), lambda i: (0, 0)),
        pl.BlockSpec((D, D), lambda i: (0, 0)),
        pl.BlockSpec((2 * D, D), lambda i: (0, 0)),
    ],
    out_specs=pl.BlockSpec((BN, D), lambda i: (i, 0)),
    out_shape=jax.ShapeDtypeStruct((NPAD, D), jnp.float32),
)


def kernel(x, edge_index, Ws, Wn_in, Wn_out, Wg):
    pad = jnp.full((EP - E,), NPAD - 1, jnp.int32)
    srcp = jnp.concatenate([edge_index[0], pad]).reshape(NW, NCHUNK, CHUNK)
    dstp = jnp.concatenate([edge_index[1], pad]).reshape(NW, NCHUNK, CHUNK)
    xp = jnp.pad(x, ((0, NPAD - N), (0, 0)))
    ain, aout, din, dout = _sc_agg_deg(xp, srcp, dstp)
    h1 = _fuse(xp, ain, aout, din, dout, Ws[0], Wn_in[0], Wn_out[0], Wg[0])
    ain2, aout2 = _sc_agg(h1, srcp, dstp)
    h2 = _fuse(h1, ain2, aout2, din, dout, Ws[1], Wn_in[1], Wn_out[1], Wg[1])
    return h2[:N]


# speedup vs baseline: 3.0384x; 3.0384x over previous
import functools

import jax
import jax.numpy as jnp
from jax import lax
from jax.experimental import pallas as pl
from jax.experimental.pallas import tpu as pltpu
from jax.experimental.pallas import tpu_sc as plsc

N = 10000
E = 320000
D = 128
NC = 2
NS = 16
NW = NC * NS
CHUNK = 80
NCHUNK = 128
EPW = NCHUNK * CHUNK
EP = NW * EPW
NPAD = 10240
RPT = NPAD // NS


def _sc_agg_body(with_deg, h_hbm, src_hbm, dst_hbm, *refs):
    if with_deg:
        (ain_hbm, aout_hbm, din_hbm, dout_hbm, acc_sh,
         si0, si1, si2, si3, di0, di1, di2, di3, rows0, rows1,
         semi0, semi1, semi2, semi3, semg0, semg1, sems0, sems1) = refs
    else:
        (ain_hbm, aout_hbm, acc_sh,
         si0, si1, si2, si3, di0, di1, di2, di3, rows0, rows1,
         semi0, semi1, semi2, semi3, semg0, semg1, sems0, sems1) = refs

    sidxs = (si0, si1, si2, si3)
    didxs = (di0, di1, di2, di3)
    rows = (rows0, rows1)
    semi = (semi0, semi1, semi2, semi3)
    semg = (semg0, semg1)
    sems = (sems0, sems1)

    cid = lax.axis_index("c")
    sid = lax.axis_index("s")
    wid = cid * NS + sid
    r0 = sid * RPT

    def _fill_rows(val):
        vv = jnp.full((16,), val, jnp.float32)

        def _fill(i, c):
            for jj in range(D // 16):
                rows0[i, pl.ds(jj * 16, 16)] = vv
            return c
        lax.fori_loop(0, CHUNK, _fill, 0)

    def _zero_acc():
        _fill_rows(0.0)
        for k in range(RPT // CHUNK):
            pltpu.sync_copy(rows0, acc_sh.at[pl.ds(r0 + k * CHUNK, CHUNK)])

    def _accumulate(gsel_hbm, ssel_hbm):
        def start_idx(k, ib):
            pltpu.async_copy(gsel_hbm.at[wid, k], sidxs[ib], semi[ib])
            pltpu.async_copy(ssel_hbm.at[wid, k], didxs[ib], semi[ib])

        def wait_idx(ib):
            pltpu.make_async_copy(gsel_hbm.at[wid, 0], sidxs[ib],
                                  semi[ib]).wait()
            pltpu.make_async_copy(ssel_hbm.at[wid, 0], didxs[ib],
                                  semi[ib]).wait()

        def start_gather(b, ib):
            pltpu.async_copy(h_hbm.at[sidxs[ib]], rows[b], semg[b])

        def wait_gather(b):
            pltpu.make_async_copy(h_hbm.at[sidxs[0]], rows[b],
                                  semg[b]).wait()

        def start_scatter(b, ib):
            pltpu.async_copy(rows[b], acc_sh.at[didxs[ib]], sems[b],
                             add=True)

        def wait_scatter(b, ib):
            pltpu.make_async_copy(rows[b], acc_sh.at[didxs[ib]],
                                  sems[b]).wait()

        def sub(k, t, with_idx=True):
            wait_idx((t + 1) % 4)
            wait_scatter((t + 1) % 2, (t + 3) % 4)
            if with_idx:
                start_idx(k + 2, (t + 2) % 4)
            start_gather((t + 1) % 2, (t + 1) % 4)
            wait_gather(t % 2)
            start_scatter(t % 2, t % 4)

        start_idx(0, 0)
        start_idx(1, 1)
        wait_idx(0)
        start_gather(0, 0)
        wait_idx(1)
        start_idx(2, 2)
        start_gather(1, 1)
        wait_gather(0)
        start_scatter(0, 0)

        def _body(i, c):
            k = 4 * i + 1
            sub(k, 1)
            sub(k + 1, 2)
            sub(k + 2, 3)
            sub(k + 3, 0)
            return c
        lax.fori_loop(0, (NCHUNK - 4) // 4, _body, 0)
        sub(NCHUNK - 3, 1)
        sub(NCHUNK - 2, 2, with_idx=False)
        wait_scatter(0, 2)
        wait_gather(1)
        start_scatter(1, 3)
        wait_scatter(1, 3)

    def _count(ssel_hbm):
        def start_idx(k, ib):
            pltpu.async_copy(ssel_hbm.at[wid, k], didxs[ib], semi[ib])

        def wait_idx(ib):
            pltpu.make_async_copy(ssel_hbm.at[wid, 0], didxs[ib],
                                  semi[ib]).wait()

        def start_scatter(b, ib):
            pltpu.async_copy(rows0, acc_sh.at[didxs[ib]], sems[b], add=True)

        def wait_scatter(b, ib):
            pltpu.make_async_copy(rows0, acc_sh.at[didxs[ib]],
                                  sems[b]).wait()

        def sub(k, t, with_idx=True):
            wait_scatter((t + 1) % 2, (t + 3) % 4)
            if with_idx:
                start_idx(k + 2, (t + 2) % 4)
            wait_idx(t % 4)
            start_scatter(t % 2, t % 4)

        start_idx(0, 0)
        start_idx(1, 1)
        start_idx(2, 2)
        wait_idx(0)
        start_scatter(0, 0)
        sub(1, 1, with_idx=True)

        def _body(i, c):
            k = 4 * i + 2
            sub(k, 2)
            sub(k + 1, 3)
            sub(k + 2, 0)
            sub(k + 3, 1)
            return c
        lax.fori_loop(0, (NCHUNK - 4) // 4, _body, 0)
        sub(NCHUNK - 2, 2, with_idx=False)
        sub(NCHUNK - 1, 3, with_idx=False)
        wait_scatter(1, 3)

    def _writeout(out_hbm):
        pltpu.sync_copy(acc_sh.at[pl.ds(r0, RPT)],
                        out_hbm.at[cid, pl.ds(r0, RPT)])

    _zero_acc()
    plsc.subcore_barrier()
    _accumulate(src_hbm, dst_hbm)
    plsc.subcore_barrier()
    _writeout(ain_hbm)
    _zero_acc()
    plsc.subcore_barrier()

    _accumulate(dst_hbm, src_hbm)
    plsc.subcore_barrier()
    _writeout(aout_hbm)

    if with_deg:
        _zero_acc()
        _fill_rows(1.0)
        plsc.subcore_barrier()
        _count(dst_hbm)
        plsc.subcore_barrier()
        _writeout(din_hbm)
        _zero_acc()
        _fill_rows(1.0)
        plsc.subcore_barrier()
        _count(src_hbm)
        plsc.subcore_barrier()
        _writeout(dout_hbm)


def _make_sc_agg(with_deg):
    n_out = 4 if with_deg else 2
    out_type = tuple(jax.ShapeDtypeStruct((NC, NPAD, D), jnp.float32)
                     for _ in range(n_out))
    scratch = ([pltpu.VMEM_SHARED((NPAD, D), jnp.float32)]
               + [pltpu.VMEM((CHUNK,), jnp.int32) for _ in range(8)]
               + [pltpu.VMEM((CHUNK, D), jnp.float32) for _ in range(2)]
               + [pltpu.SemaphoreType.DMA for _ in range(8)])
    mesh = plsc.VectorSubcoreMesh(core_axis_name="c", subcore_axis_name="s")
    return pl.kernel(functools.partial(_sc_agg_body, with_deg),
                     out_type=out_type, mesh=mesh,
                     scratch_types=scratch)


_sc_agg_deg = _make_sc_agg(True)
_sc_agg = _make_sc_agg(False)

BN = 512
GRID = NPAD // BN


def _fuse_body(h_ref, ain_ref, aout_ref, din_ref, dout_ref,
               ws_ref, wni_ref, wno_ref, wg_ref, out_ref):
    f32 = jnp.float32
    h = h_ref[...]
    din = jnp.maximum(din_ref[0, :, 0:1] + din_ref[1, :, 0:1], 1.0)
    dout = jnp.maximum(dout_ref[0, :, 0:1] + dout_ref[1, :, 0:1], 1.0)
    a_in = (ain_ref[0] + ain_ref[1]) / din
    a_out = (aout_ref[0] + aout_ref[1]) / dout
    hs = jnp.dot(h, ws_ref[...], preferred_element_type=f32)
    h_in = hs + jnp.dot(a_in, wni_ref[...], preferred_element_type=f32)
    h_out = hs + jnp.dot(a_out, wno_ref[...], preferred_element_type=f32)
    g = (jnp.dot(h_in, wg_ref[0:D, :], preferred_element_type=f32)
         + jnp.dot(h_out, wg_ref[D:2 * D, :], preferred_element_type=f32))
    z = jax.nn.sigmoid(g)
    out_ref[...] = jnp.maximum(z * h_in + (1.0 - z) * h_out, 0.0)


_fuse = pl.pallas_call(
    _fuse_body,
    grid=(GRID,),
    in_specs=[
        pl.BlockSpec((BN, D), lambda i: (i, 0)),
        pl.BlockSpec((NC, BN, D), lambda i: (0, i, 0)),
        pl.BlockSpec((NC, BN, D), lambda i: (0, i, 0)),
        pl.BlockSpec((NC, BN, D), lambda i: (0, i, 0)),
        pl.BlockSpec((NC, BN, D), lambda i: (0, i, 0)),
        pl.BlockSpec((D, D), lambda i: (0, 0)),
        pl.BlockSpec((D, D), lambda i: (0, 0)),
        pl.BlockSpec((D, D), lambda i: (0, 0)),
        pl.BlockSpec((2 * D, D), lambda i: (0, 0)),
    ],
    out_specs=pl.BlockSpec((BN, D), lambda i: (i, 0)),
    out_shape=jax.ShapeDtypeStruct((NPAD, D), jnp.float32),
)


def kernel(x, edge_index, Ws, Wn_in, Wn_out, Wg):
    pad = N + (jnp.arange(EP - E, dtype=jnp.int32) % (NPAD - N))
    srcp = jnp.concatenate([edge_index[0], pad]).reshape(NW, NCHUNK, CHUNK)
    dstp = jnp.concatenate([edge_index[1], pad]).reshape(NW, NCHUNK, CHUNK)
    xp = jnp.pad(x, ((0, NPAD - N), (0, 0)))
    ain, aout, din, dout = _sc_agg_deg(xp, srcp, dstp)
    h1 = _fuse(xp, ain, aout, din, dout, Ws[0], Wn_in[0], Wn_out[0], Wg[0])
    ain2, aout2 = _sc_agg(h1, srcp, dstp)
    h2 = _fuse(h1, ain2, aout2, din, dout, Ws[1], Wn_in[1], Wn_out[1], Wg[1])
    return h2[:N]


# speedup vs baseline: 3.1499x; 1.0367x over previous
import functools

import jax
import jax.numpy as jnp
from jax import lax
from jax.experimental import pallas as pl
from jax.experimental.pallas import tpu as pltpu
from jax.experimental.pallas import tpu_sc as plsc

N = 10000
E = 320000
D = 128
NC = 2
NS = 16
NW = NC * NS
CHUNK = 96
NCHUNK = 108
EPW = NCHUNK * CHUNK
EP = NW * EPW
NPAD = 10240
RPT = NPAD // NS


def _sc_agg_body(with_deg, h_hbm, src_hbm, dst_hbm, *refs):
    if with_deg:
        (ain_hbm, aout_hbm, din_hbm, dout_hbm, acc_sh,
         si0, si1, si2, si3, di0, di1, di2, di3, rows0, rows1,
         semi0, semi1, semi2, semi3, semg0, semg1, sems0, sems1) = refs
    else:
        (ain_hbm, aout_hbm, acc_sh,
         si0, si1, si2, si3, di0, di1, di2, di3, rows0, rows1,
         semi0, semi1, semi2, semi3, semg0, semg1, sems0, sems1) = refs

    sidxs = (si0, si1, si2, si3)
    didxs = (di0, di1, di2, di3)
    rows = (rows0, rows1)
    semi = (semi0, semi1, semi2, semi3)
    semg = (semg0, semg1)
    sems = (sems0, sems1)

    cid = lax.axis_index("c")
    sid = lax.axis_index("s")
    wid = cid * NS + sid
    r0 = sid * RPT

    def _fill_rows(val):
        vv = jnp.full((16,), val, jnp.float32)

        def _fill(i, c):
            for jj in range(D // 16):
                rows0[i, pl.ds(jj * 16, 16)] = vv
            return c
        lax.fori_loop(0, CHUNK, _fill, 0)

    def _zero_acc():
        _fill_rows(0.0)
        for k in range(RPT // CHUNK):
            pltpu.sync_copy(rows0, acc_sh.at[pl.ds(r0 + k * CHUNK, CHUNK)])
        rem = RPT - (RPT // CHUNK) * CHUNK
        if rem:
            pltpu.sync_copy(rows0.at[pl.ds(0, rem)],
                            acc_sh.at[pl.ds(r0 + (RPT // CHUNK) * CHUNK, rem)])

    def _accumulate(gsel_hbm, ssel_hbm):
        def start_idx(k, ib):
            pltpu.async_copy(gsel_hbm.at[wid, k], sidxs[ib], semi[ib])
            pltpu.async_copy(ssel_hbm.at[wid, k], didxs[ib], semi[ib])

        def wait_idx(ib):
            pltpu.make_async_copy(gsel_hbm.at[wid, 0], sidxs[ib],
                                  semi[ib]).wait()
            pltpu.make_async_copy(ssel_hbm.at[wid, 0], didxs[ib],
                                  semi[ib]).wait()

        def start_gather(b, ib):
            pltpu.async_copy(h_hbm.at[sidxs[ib]], rows[b], semg[b])

        def wait_gather(b):
            pltpu.make_async_copy(h_hbm.at[sidxs[0]], rows[b],
                                  semg[b]).wait()

        def start_scatter(b, ib):
            pltpu.async_copy(rows[b], acc_sh.at[didxs[ib]], sems[b],
                             add=True)

        def wait_scatter(b, ib):
            pltpu.make_async_copy(rows[b], acc_sh.at[didxs[ib]],
                                  sems[b]).wait()

        def sub(k, t, with_idx=True):
            wait_idx((t + 1) % 4)
            wait_scatter((t + 1) % 2, (t + 3) % 4)
            if with_idx:
                start_idx(k + 2, (t + 2) % 4)
            start_gather((t + 1) % 2, (t + 1) % 4)
            wait_gather(t % 2)
            start_scatter(t % 2, t % 4)

        start_idx(0, 0)
        start_idx(1, 1)
        wait_idx(0)
        start_gather(0, 0)
        wait_idx(1)
        start_idx(2, 2)
        start_gather(1, 1)
        wait_gather(0)
        start_scatter(0, 0)

        def _body(i, c):
            k = 4 * i + 1
            sub(k, 1)
            sub(k + 1, 2)
            sub(k + 2, 3)
            sub(k + 3, 0)
            return c
        lax.fori_loop(0, (NCHUNK - 4) // 4, _body, 0)
        sub(NCHUNK - 3, 1)
        sub(NCHUNK - 2, 2, with_idx=False)
        wait_scatter(0, 2)
        wait_gather(1)
        start_scatter(1, 3)
        wait_scatter(1, 3)

    def _count(ssel_hbm):
        def start_idx(k, ib):
            pltpu.async_copy(ssel_hbm.at[wid, k], didxs[ib], semi[ib])

        def wait_idx(ib):
            pltpu.make_async_copy(ssel_hbm.at[wid, 0], didxs[ib],
                                  semi[ib]).wait()

        def start_scatter(b, ib):
            pltpu.async_copy(rows0, acc_sh.at[didxs[ib]], sems[b], add=True)

        def wait_scatter(b, ib):
            pltpu.make_async_copy(rows0, acc_sh.at[didxs[ib]],
                                  sems[b]).wait()

        def sub(k, t, with_idx=True):
            wait_scatter((t + 1) % 2, (t + 3) % 4)
            if with_idx:
                start_idx(k + 2, (t + 2) % 4)
            wait_idx(t % 4)
            start_scatter(t % 2, t % 4)

        start_idx(0, 0)
        start_idx(1, 1)
        start_idx(2, 2)
        wait_idx(0)
        start_scatter(0, 0)
        sub(1, 1, with_idx=True)

        def _body(i, c):
            k = 4 * i + 2
            sub(k, 2)
            sub(k + 1, 3)
            sub(k + 2, 0)
            sub(k + 3, 1)
            return c
        lax.fori_loop(0, (NCHUNK - 4) // 4, _body, 0)
        sub(NCHUNK - 2, 2, with_idx=False)
        sub(NCHUNK - 1, 3, with_idx=False)
        wait_scatter(1, 3)

    def _writeout(out_hbm):
        pltpu.sync_copy(acc_sh.at[pl.ds(r0, RPT)],
                        out_hbm.at[cid, pl.ds(r0, RPT)])

    _zero_acc()
    plsc.subcore_barrier()
    _accumulate(src_hbm, dst_hbm)
    plsc.subcore_barrier()
    _writeout(ain_hbm)
    _zero_acc()
    plsc.subcore_barrier()

    _accumulate(dst_hbm, src_hbm)
    plsc.subcore_barrier()
    _writeout(aout_hbm)

    if with_deg:
        _zero_acc()
        _fill_rows(1.0)
        plsc.subcore_barrier()
        _count(dst_hbm)
        plsc.subcore_barrier()
        _writeout(din_hbm)
        _zero_acc()
        _fill_rows(1.0)
        plsc.subcore_barrier()
        _count(src_hbm)
        plsc.subcore_barrier()
        _writeout(dout_hbm)


def _make_sc_agg(with_deg):
    n_out = 4 if with_deg else 2
    out_type = tuple(jax.ShapeDtypeStruct((NC, NPAD, D), jnp.float32)
                     for _ in range(n_out))
    scratch = ([pltpu.VMEM_SHARED((NPAD, D), jnp.float32)]
               + [pltpu.VMEM((CHUNK,), jnp.int32) for _ in range(8)]
               + [pltpu.VMEM((CHUNK, D), jnp.float32) for _ in range(2)]
               + [pltpu.SemaphoreType.DMA for _ in range(8)])
    mesh = plsc.VectorSubcoreMesh(core_axis_name="c", subcore_axis_name="s")
    return pl.kernel(functools.partial(_sc_agg_body, with_deg),
                     out_type=out_type, mesh=mesh,
                     scratch_types=scratch)


_sc_agg_deg = _make_sc_agg(True)
_sc_agg = _make_sc_agg(False)

BN = 512
GRID = NPAD // BN


def _fuse_body(h_ref, ain_ref, aout_ref, din_ref, dout_ref,
               ws_ref, wni_ref, wno_ref, wg_ref, out_ref):
    f32 = jnp.float32
    h = h_ref[...]
    din = jnp.maximum(din_ref[0, :, 0:1] + din_ref[1, :, 0:1], 1.0)
    dout = jnp.maximum(dout_ref[0, :, 0:1] + dout_ref[1, :, 0:1], 1.0)
    a_in = (ain_ref[0] + ain_ref[1]) / din
    a_out = (aout_ref[0] + aout_ref[1]) / dout
    hs = jnp.dot(h, ws_ref[...], preferred_element_type=f32)
    h_in = hs + jnp.dot(a_in, wni_ref[...], preferred_element_type=f32)
    h_out = hs + jnp.dot(a_out, wno_ref[...], preferred_element_type=f32)
    g = (jnp.dot(h_in, wg_ref[0:D, :], preferred_element_type=f32)
         + jnp.dot(h_out, wg_ref[D:2 * D, :], preferred_element_type=f32))
    z = jax.nn.sigmoid(g)
    out_ref[...] = jnp.maximum(z * h_in + (1.0 - z) * h_out, 0.0)


_fuse_in_specs = [
        pl.BlockSpec((BN, D), lambda i: (i, 0)),
        pl.BlockSpec((NC, BN, D), lambda i: (0, i, 0)),
        pl.BlockSpec((NC, BN, D), lambda i: (0, i, 0)),
        pl.BlockSpec((NC, BN, D), lambda i: (0, i, 0)),
        pl.BlockSpec((NC, BN, D), lambda i: (0, i, 0)),
        pl.BlockSpec((D, D), lambda i: (0, 0)),
        pl.BlockSpec((D, D), lambda i: (0, 0)),
        pl.BlockSpec((D, D), lambda i: (0, 0)),
        pl.BlockSpec((2 * D, D), lambda i: (0, 0)),
]

_fuse = pl.pallas_call(
    _fuse_body,
    grid=(GRID,),
    in_specs=_fuse_in_specs,
    out_specs=pl.BlockSpec((BN, D), lambda i: (i, 0)),
    out_shape=jax.ShapeDtypeStruct((NPAD, D), jnp.float32),
)

_fuse_last = pl.pallas_call(
    _fuse_body,
    grid=(GRID,),
    in_specs=_fuse_in_specs,
    out_specs=pl.BlockSpec((BN, D), lambda i: (i, 0)),
    out_shape=jax.ShapeDtypeStruct((N, D), jnp.float32),
)


def kernel(x, edge_index, Ws, Wn_in, Wn_out, Wg):
    pad = N + (jnp.arange(EP - E, dtype=jnp.int32) % (NPAD - N))
    srcp = jnp.concatenate([edge_index[0], pad]).reshape(NW, NCHUNK, CHUNK)
    dstp = jnp.concatenate([edge_index[1], pad]).reshape(NW, NCHUNK, CHUNK)
    xp = jnp.pad(x, ((0, NPAD - N), (0, 0)))
    ain, aout, din, dout = _sc_agg_deg(xp, srcp, dstp)
    h1 = _fuse(xp, ain, aout, din, dout, Ws[0], Wn_in[0], Wn_out[0], Wg[0])
    ain2, aout2 = _sc_agg(h1, srcp, dstp)
    return _fuse_last(h1, ain2, aout2, din, dout,
                      Ws[1], Wn_in[1], Wn_out[1], Wg[1])


# speedup vs baseline: 3.2335x; 1.0266x over previous
import functools

import jax
import jax.numpy as jnp
from jax import lax
from jax.experimental import pallas as pl
from jax.experimental.pallas import tpu as pltpu
from jax.experimental.pallas import tpu_sc as plsc

N = 10000
E = 320000
D = 128
NC = 2
NS = 16
NW = NC * NS
CHUNK = 112
NCHUNK = 92
EPW = NCHUNK * CHUNK
EP = NW * EPW
NPAD = 10240
RPT = NPAD // NS


def _sc_agg_body(with_deg, h_hbm, src_hbm, dst_hbm, *refs):
    if with_deg:
        (ain_hbm, aout_hbm, din_hbm, dout_hbm, acc_sh,
         si0, si1, si2, si3, di0, di1, di2, di3, rows0, rows1,
         semi0, semi1, semi2, semi3, semg0, semg1, sems0, sems1) = refs
    else:
        (ain_hbm, aout_hbm, acc_sh,
         si0, si1, si2, si3, di0, di1, di2, di3, rows0, rows1,
         semi0, semi1, semi2, semi3, semg0, semg1, sems0, sems1) = refs

    sidxs = (si0, si1, si2, si3)
    didxs = (di0, di1, di2, di3)
    rows = (rows0, rows1)
    semi = (semi0, semi1, semi2, semi3)
    semg = (semg0, semg1)
    sems = (sems0, sems1)

    cid = lax.axis_index("c")
    sid = lax.axis_index("s")
    wid = cid * NS + sid
    r0 = sid * RPT

    def _fill_rows(val):
        vv = jnp.full((16,), val, jnp.float32)

        def _fill(i, c):
            for jj in range(D // 16):
                rows0[i, pl.ds(jj * 16, 16)] = vv
            return c
        lax.fori_loop(0, CHUNK, _fill, 0)

    def _zero_acc():
        _fill_rows(0.0)
        for k in range(RPT // CHUNK):
            pltpu.sync_copy(rows0, acc_sh.at[pl.ds(r0 + k * CHUNK, CHUNK)])
        rem = RPT - (RPT // CHUNK) * CHUNK
        if rem:
            pltpu.sync_copy(rows0.at[pl.ds(0, rem)],
                            acc_sh.at[pl.ds(r0 + (RPT // CHUNK) * CHUNK, rem)])

    def _accumulate(gsel_hbm, ssel_hbm):
        def start_idx(k, ib):
            pltpu.async_copy(gsel_hbm.at[wid, k], sidxs[ib], semi[ib])
            pltpu.async_copy(ssel_hbm.at[wid, k], didxs[ib], semi[ib])

        def wait_idx(ib):
            pltpu.make_async_copy(gsel_hbm.at[wid, 0], sidxs[ib],
                                  semi[ib]).wait()
            pltpu.make_async_copy(ssel_hbm.at[wid, 0], didxs[ib],
                                  semi[ib]).wait()

        def start_gather(b, ib):
            pltpu.async_copy(h_hbm.at[sidxs[ib]], rows[b], semg[b])

        def wait_gather(b):
            pltpu.make_async_copy(h_hbm.at[sidxs[0]], rows[b],
                                  semg[b]).wait()

        def start_scatter(b, ib):
            pltpu.async_copy(rows[b], acc_sh.at[didxs[ib]], sems[b],
                             add=True)

        def wait_scatter(b, ib):
            pltpu.make_async_copy(rows[b], acc_sh.at[didxs[ib]],
                                  sems[b]).wait()

        def sub(k, t, with_idx=True):
            wait_idx((t + 1) % 4)
            wait_scatter((t + 1) % 2, (t + 3) % 4)
            if with_idx:
                start_idx(k + 2, (t + 2) % 4)
            start_gather((t + 1) % 2, (t + 1) % 4)
            wait_gather(t % 2)
            start_scatter(t % 2, t % 4)

        start_idx(0, 0)
        start_idx(1, 1)
        wait_idx(0)
        start_gather(0, 0)
        wait_idx(1)
        start_idx(2, 2)
        start_gather(1, 1)
        wait_gather(0)
        start_scatter(0, 0)

        def _body(i, c):
            k = 4 * i + 1
            sub(k, 1)
            sub(k + 1, 2)
            sub(k + 2, 3)
            sub(k + 3, 0)
            return c
        lax.fori_loop(0, (NCHUNK - 4) // 4, _body, 0)
        sub(NCHUNK - 3, 1)
        sub(NCHUNK - 2, 2, with_idx=False)
        wait_scatter(0, 2)
        wait_gather(1)
        start_scatter(1, 3)
        wait_scatter(1, 3)

    def _count(ssel_hbm):
        def start_idx(k, ib):
            pltpu.async_copy(ssel_hbm.at[wid, k], didxs[ib], semi[ib])

        def wait_idx(ib):
            pltpu.make_async_copy(ssel_hbm.at[wid, 0], didxs[ib],
                                  semi[ib]).wait()

        def start_scatter(b, ib):
            pltpu.async_copy(rows0, acc_sh.at[didxs[ib]], sems[b], add=True)

        def wait_scatter(b, ib):
            pltpu.make_async_copy(rows0, acc_sh.at[didxs[ib]],
                                  sems[b]).wait()

        def sub(k, t, with_idx=True):
            wait_scatter((t + 1) % 2, (t + 3) % 4)
            if with_idx:
                start_idx(k + 2, (t + 2) % 4)
            wait_idx(t % 4)
            start_scatter(t % 2, t % 4)

        start_idx(0, 0)
        start_idx(1, 1)
        start_idx(2, 2)
        wait_idx(0)
        start_scatter(0, 0)
        sub(1, 1, with_idx=True)

        def _body(i, c):
            k = 4 * i + 2
            sub(k, 2)
            sub(k + 1, 3)
            sub(k + 2, 0)
            sub(k + 3, 1)
            return c
        lax.fori_loop(0, (NCHUNK - 4) // 4, _body, 0)
        sub(NCHUNK - 2, 2, with_idx=False)
        sub(NCHUNK - 1, 3, with_idx=False)
        wait_scatter(1, 3)

    def _writeout(out_hbm):
        pltpu.sync_copy(acc_sh.at[pl.ds(r0, RPT)],
                        out_hbm.at[cid, pl.ds(r0, RPT)])

    _zero_acc()
    plsc.subcore_barrier()
    _accumulate(src_hbm, dst_hbm)
    plsc.subcore_barrier()
    _writeout(ain_hbm)
    _zero_acc()
    plsc.subcore_barrier()

    _accumulate(dst_hbm, src_hbm)
    plsc.subcore_barrier()
    _writeout(aout_hbm)

    if with_deg:
        _zero_acc()
        _fill_rows(1.0)
        plsc.subcore_barrier()
        _count(dst_hbm)
        plsc.subcore_barrier()
        _writeout(din_hbm)
        _zero_acc()
        _fill_rows(1.0)
        plsc.subcore_barrier()
        _count(src_hbm)
        plsc.subcore_barrier()
        _writeout(dout_hbm)


def _make_sc_agg(with_deg):
    n_out = 4 if with_deg else 2
    out_type = tuple(jax.ShapeDtypeStruct((NC, NPAD, D), jnp.float32)
                     for _ in range(n_out))
    scratch = ([pltpu.VMEM_SHARED((NPAD, D), jnp.float32)]
               + [pltpu.VMEM((CHUNK,), jnp.int32) for _ in range(8)]
               + [pltpu.VMEM((CHUNK, D), jnp.float32) for _ in range(2)]
               + [pltpu.SemaphoreType.DMA for _ in range(8)])
    mesh = plsc.VectorSubcoreMesh(core_axis_name="c", subcore_axis_name="s")
    return pl.kernel(functools.partial(_sc_agg_body, with_deg),
                     out_type=out_type, mesh=mesh,
                     scratch_types=scratch)


_sc_agg_deg = _make_sc_agg(True)
_sc_agg = _make_sc_agg(False)

BN = 512
GRID = NPAD // BN


def _fuse_body(h_ref, ain_ref, aout_ref, din_ref, dout_ref,
               ws_ref, wni_ref, wno_ref, wg_ref, out_ref):
    f32 = jnp.float32
    h = h_ref[...]
    din = jnp.maximum(din_ref[0, :, 0:1] + din_ref[1, :, 0:1], 1.0)
    dout = jnp.maximum(dout_ref[0, :, 0:1] + dout_ref[1, :, 0:1], 1.0)
    a_in = (ain_ref[0] + ain_ref[1]) / din
    a_out = (aout_ref[0] + aout_ref[1]) / dout
    hs = jnp.dot(h, ws_ref[...], preferred_element_type=f32)
    h_in = hs + jnp.dot(a_in, wni_ref[...], preferred_element_type=f32)
    h_out = hs + jnp.dot(a_out, wno_ref[...], preferred_element_type=f32)
    g = (jnp.dot(h_in, wg_ref[0:D, :], preferred_element_type=f32)
         + jnp.dot(h_out, wg_ref[D:2 * D, :], preferred_element_type=f32))
    z = jax.nn.sigmoid(g)
    out_ref[...] = jnp.maximum(z * h_in + (1.0 - z) * h_out, 0.0)


_fuse_in_specs = [
        pl.BlockSpec((BN, D), lambda i: (i, 0)),
        pl.BlockSpec((NC, BN, D), lambda i: (0, i, 0)),
        pl.BlockSpec((NC, BN, D), lambda i: (0, i, 0)),
        pl.BlockSpec((NC, BN, D), lambda i: (0, i, 0)),
        pl.BlockSpec((NC, BN, D), lambda i: (0, i, 0)),
        pl.BlockSpec((D, D), lambda i: (0, 0)),
        pl.BlockSpec((D, D), lambda i: (0, 0)),
        pl.BlockSpec((D, D), lambda i: (0, 0)),
        pl.BlockSpec((2 * D, D), lambda i: (0, 0)),
]

_fuse = pl.pallas_call(
    _fuse_body,
    grid=(GRID,),
    in_specs=_fuse_in_specs,
    out_specs=pl.BlockSpec((BN, D), lambda i: (i, 0)),
    out_shape=jax.ShapeDtypeStruct((NPAD, D), jnp.float32),
)

_fuse_last = pl.pallas_call(
    _fuse_body,
    grid=(GRID,),
    in_specs=_fuse_in_specs,
    out_specs=pl.BlockSpec((BN, D), lambda i: (i, 0)),
    out_shape=jax.ShapeDtypeStruct((N, D), jnp.float32),
)


def kernel(x, edge_index, Ws, Wn_in, Wn_out, Wg):
    pad = N + (jnp.arange(EP - E, dtype=jnp.int32) % (NPAD - N))
    srcp = jnp.concatenate([edge_index[0], pad]).reshape(NW, NCHUNK, CHUNK)
    dstp = jnp.concatenate([edge_index[1], pad]).reshape(NW, NCHUNK, CHUNK)
    xp = jnp.pad(x, ((0, NPAD - N), (0, 0)))
    ain, aout, din, dout = _sc_agg_deg(xp, srcp, dstp)
    h1 = _fuse(xp, ain, aout, din, dout, Ws[0], Wn_in[0], Wn_out[0], Wg[0])
    ain2, aout2 = _sc_agg(h1, srcp, dstp)
    return _fuse_last(h1, ain2, aout2, din, dout,
                      Ws[1], Wn_in[1], Wn_out[1], Wg[1])
